# baseline (speedup 1.0000x reference)
import functools
import jax
import jax.numpy as jnp
from jax import lax
from jax.experimental import pallas as pl
from jax.experimental.pallas import tpu as pltpu
from jax.experimental.pallas import tpu_sc as plsc

N = 10000
E = 320000
FIN = 128
HID = 256
NB = 64
NC = 2560
NT = 32
RPT = 320
EPT = E // NT
CHA = 2000
CHB = 4000
CAP = E + 16384
G = 128
SENT = RPT << 14

@functools.cache
def _mesh():
    return plsc.VectorSubcoreMesh(core_axis_name="c", subcore_axis_name="s")


def _s1_body(src_hbm, dst_hbm, clus_hbm, elist, counts, indeg, psrc, pdst,
             sbuf0, sbuf1, dbuf0, dbuf1, packbuf, abuf_s, abuf_d, pbs, pbd,
             clus_v, hist, indeg_v, cnt_v, sem_b0, sem_b1):
    c = lax.axis_index("c")
    s = lax.axis_index("s")
    wid = s * 2 + c
    lane = lax.iota(jnp.int32, 16)
    ones16 = jnp.ones((16,), jnp.float32)
    sem_b = (sem_b0, sem_b1)
    sbufs = (sbuf0, sbuf1)
    dbufs = (dbuf0, dbuf1)

    pltpu.sync_copy(clus_hbm, clus_v)
    ebase = wid * EPT

    def chunk_a(a, _):
        off = pl.multiple_of(ebase + a * CHA, 8)
        pltpu.sync_copy(src_hbm.at[pl.ds(off, CHA)], abuf_s)
        pltpu.sync_copy(dst_hbm.at[pl.ds(off, CHA)], abuf_d)

        def va(v, _):
            s16 = abuf_s[pl.ds(v * 16, 16)]
            d16 = abuf_d[pl.ds(v * 16, 16)]
            pbs[pl.ds(v * 16, 16)] = plsc.load_gather(clus_v, [s16])
            pbd[pl.ds(v * 16, 16)] = plsc.load_gather(clus_v, [d16])
            return 0

        lax.fori_loop(0, CHA // 16, va, 0)
        pltpu.sync_copy(pbs, psrc.at[pl.ds(off, CHA)])
        pltpu.sync_copy(pbd, pdst.at[pl.ds(off, CHA)])
        return 0

    lax.fori_loop(0, EPT // CHA, chunk_a, 0)

    def zh(i, _):
        hist[pl.ds(i * 16, 16)] = jnp.zeros((16,), jnp.float32)
        return 0

    lax.fori_loop(0, (RPT * 16) // 16, zh, 0)

    lo = wid * RPT
    hi = lo + RPT

    def fetch_b(ch, p):
        off = pl.multiple_of(ch * CHB, 8)
        pltpu.async_copy(src_hbm.at[pl.ds(off, CHB)], sbufs[p], sem_b[p])
        pltpu.async_copy(dst_hbm.at[pl.ds(off, CHB)], dbufs[p], sem_b[p])

    def proc_b(ch, p, gbase):
        off = pl.multiple_of(ch * CHB, 8)
        pltpu.make_async_copy(src_hbm.at[pl.ds(off, CHB)], sbufs[p],
                              sem_b[p]).wait()
        pltpu.make_async_copy(dst_hbm.at[pl.ds(off, CHB)], dbufs[p],
                              sem_b[p]).wait()

        def vb(v, lcnt):
            s16 = sbufs[p][pl.ds(v * 16, 16)]
            d16 = dbufs[p][pl.ds(v * 16, 16)]
            m = (d16 >= lo) & (d16 < hi)
            ld16 = d16 - lo
            packed = s16 | (ld16 << 14)
            plsc.store_compressed(packbuf.at[pl.ds(lcnt, 16)], packed, mask=m)
            hidx = jnp.where(m, lane * RPT + ld16, lane * RPT)
            plsc.addupdate_scatter(hist, [hidx], ones16, mask=m)
            return lcnt + plsc.all_reduce_population_count(m)[0]

        lcnt = lax.fori_loop(0, CHB // 16, vb, jnp.int32(0))
        pad = (8 - (lcnt & 7)) & 7
        pm = lane < pad
        plsc.store_compressed(packbuf.at[pl.ds(lcnt, 16)],
                              jnp.full((16,), SENT, jnp.int32), mask=pm)
        pltpu.sync_copy(packbuf,
                        elist.at[pl.ds(pl.multiple_of(wid * CAP + gbase, 8),
                                       CHB + 16)])
        return gbase + lcnt + pad

    NPB = E // CHB // 2

    fetch_b(0, 0)

    def pair_b(q, gbase):
        fetch_b(2 * q + 1, 1)
        gbase = proc_b(2 * q, 0, gbase)

        @pl.when(q < NPB - 1)
        def _():
            fetch_b(2 * q + 2, 0)

        return proc_b(2 * q + 1, 1, gbase)

    gfinal = lax.fori_loop(0, NPB, pair_b, jnp.int32(0))

    def fs(i, _):
        packbuf[pl.ds(i * 16, 16)] = jnp.full((16,), SENT, jnp.int32)
        return 0

    lax.fori_loop(0, (CHB + 16) // 16, fs, 0)
    pltpu.sync_copy(packbuf,
                    elist.at[pl.ds(pl.multiple_of(wid * CAP + gfinal, 8),
                                   CHB + 16)])
    pltpu.sync_copy(packbuf,
                    elist.at[pl.ds(pl.multiple_of(
                        wid * CAP + gfinal + CHB + 16, 8), CHB + 16)])

    cnt_v[...] = jnp.full((16,), gfinal, jnp.int32)
    pltpu.sync_copy(cnt_v.at[pl.ds(0, 8)],
                    counts.at[pl.ds(pl.multiple_of(wid * 8, 8), 8)])

    def hr(v, _):
        acc = hist[pl.ds(v * 16, 16)]
        for l in range(1, 16):
            acc = acc + hist[pl.ds(l * RPT + v * 16, 16)]
        indeg_v[pl.ds(v * 16, 16)] = acc
        return 0

    lax.fori_loop(0, RPT // 16, hr, 0)
    pltpu.sync_copy(indeg_v,
                    indeg.at[pl.ds(pl.multiple_of(wid * RPT, 8), RPT)])


def _s1_call(src, dst, cluster):
    f = pl.kernel(
        _s1_body,
        out_type=(
            jax.ShapeDtypeStruct((NT * CAP,), jnp.int32),
            jax.ShapeDtypeStruct((NT * 8,), jnp.int32),
            jax.ShapeDtypeStruct((NT * RPT,), jnp.float32),
            jax.ShapeDtypeStruct((E,), jnp.int32),
            jax.ShapeDtypeStruct((E,), jnp.int32),
        ),
        mesh=_mesh(),
        compiler_params=pltpu.CompilerParams(needs_layout_passes=False),
        scratch_types=[
            pltpu.VMEM((CHB,), jnp.int32),
            pltpu.VMEM((CHB,), jnp.int32),
            pltpu.VMEM((CHB,), jnp.int32),
            pltpu.VMEM((CHB,), jnp.int32),
            pltpu.VMEM((CHB + 16,), jnp.int32),
            pltpu.VMEM((CHA,), jnp.int32),
            pltpu.VMEM((CHA,), jnp.int32),
            pltpu.VMEM((CHA,), jnp.int32),
            pltpu.VMEM((CHA,), jnp.int32),
            pltpu.VMEM((N,), jnp.int32),
            pltpu.VMEM((RPT * 16,), jnp.float32),
            pltpu.VMEM((RPT,), jnp.float32),
            pltpu.VMEM((16,), jnp.int32),
            pltpu.SemaphoreType.DMA,
            pltpu.SemaphoreType.DMA,
        ],
        name="s1_edge_scan",
    )
    return f(src, dst, cluster)


ACCW = (RPT + 8) * FIN


def _s2_body(ux_hbm, elist, counts, aggx, acc, lbig, idxs, lds,
             rows00, cnt_v, sem00):
    c = lax.axis_index("c")
    s = lax.axis_index("s")
    wid = s * 2 + c
    sems = ((sem00,),)
    rows = ((rows00,),)

    def za(i, _):
        acc[pl.ds(i * 16, 16)] = jnp.zeros((16,), jnp.float32)
        return 0

    lax.fori_loop(0, ACCW // 16, za, 0)

    pltpu.sync_copy(counts.at[pl.ds(pl.multiple_of(wid * 8, 8), 8)],
                    cnt_v.at[pl.ds(0, 8)])
    cnt = cnt_v[pl.ds(0, 16)][0]

    LBLK = 4096
    nblk = (cnt + (LBLK - 1)) // LBLK

    def blk(b, _):
        base = wid * CAP + b * LBLK
        pltpu.sync_copy(
            elist.at[pl.ds(pl.multiple_of(base, 8), LBLK)], lbig)

        def unp(r, _):
            for jj in range(8):
                e16 = lbig[pl.ds(r * 128 + jj * 16, 16)]
                idxs[r, pl.ds(jj * 16, 16)] = e16 & 0x3FFF
                lds[r, pl.ds(jj * 16, 16)] = (e16 >> 7) & 0x7FFF80
            return 0

        lax.fori_loop(0, LBLK // 128, unp, 0)

        def sub(k, _):
            pltpu.async_copy(ux_hbm.at[idxs.at[k]], rows[0][0],
                             sems[0][0]).wait()
            rbuf = rows[0][0]

            def edge(v, _):
                ld16 = lds[k, pl.ds(v * 16, 16)]
                for kk in range(16):
                    base = ld16[kk]
                    i = v * 16 + kk
                    rs = [rbuf[i, pl.ds(j * 16, 16)]
                          for j in range(FIN // 16)]
                    accs = [acc[pl.ds(base + j * 16, 16)]
                            for j in range(FIN // 16)]
                    for j in range(FIN // 16):
                        acc[pl.ds(base + j * 16, 16)] = accs[j] + rs[j]
                return 0

            lax.fori_loop(0, G // 16, edge, 0)
            return 0

        lax.fori_loop(0, LBLK // G, sub, 0)
        return 0

    lax.fori_loop(0, nblk, blk, 0)

    pltpu.sync_copy(acc.at[pl.ds(0, RPT * FIN)],
                    aggx.at[pl.ds(pl.multiple_of(wid * RPT * FIN, 8),
                                  RPT * FIN)])


def _s2_call(ux, elist, counts):
    f = pl.kernel(
        _s2_body,
        out_type=jax.ShapeDtypeStruct((NT * RPT * FIN,), jnp.float32),
        mesh=_mesh(),
        compiler_params=pltpu.CompilerParams(needs_layout_passes=False),
        scratch_types=[
            pltpu.VMEM((ACCW,), jnp.float32),
            pltpu.VMEM((4096,), jnp.int32),
            pltpu.VMEM((32, 128), jnp.int32),
            pltpu.VMEM((32, 128), jnp.int32),
            pltpu.VMEM((G, FIN), jnp.float32),
            pltpu.VMEM((16,), jnp.int32),
            pltpu.SemaphoreType.DMA,
        ],
        name="s2_gather_acc",
    )
    return f(ux, elist, counts)


BANDR = 320
BANDW = BANDR * NC
EPS = E // 16
CHS = 4000
ZW = 6400
SHARE = BANDW // 16


def _s3_body(psrc_hbm, pdst_hbm, adj, pbuf, dbuf, fidx, vvals, zbuf, stage,
             sband):
    c = lax.axis_index("c")
    s = lax.axis_index("s")

    def zz(i, _):
        zbuf[pl.ds(i * 16, 16)] = jnp.zeros((16,), jnp.float32)
        return 0

    lax.fori_loop(0, ZW // 16, zz, 0)

    for b in range(4):
        band = c * 4 + b
        lo = band * BANDR

        def zb(i, _):
            pltpu.sync_copy(
                zbuf,
                sband.at[pl.ds(pl.multiple_of(s * SHARE + i * ZW, 8), ZW)])
            return 0

        lax.fori_loop(0, SHARE // ZW, zb, 0)
        plsc.subcore_barrier()

        ebase = s * EPS

        def chunk(ch, _):
            off = pl.multiple_of(ebase + ch * CHS, 8)
            pltpu.sync_copy(psrc_hbm.at[pl.ds(off, CHS)], pbuf)
            pltpu.sync_copy(pdst_hbm.at[pl.ds(off, CHS)], dbuf)

            def vv(vi, _):
                ps = pbuf[pl.ds(vi * 16, 16)]
                pd = dbuf[pl.ds(vi * 16, 16)]
                m = (pd >= lo) & (pd < lo + BANDR)
                flat = jnp.where(m, (pd - lo) * NC + ps, ps)
                val = jnp.where(m, 1.0, 0.0).astype(jnp.float32)
                row = vi // 8
                col = (vi % 8) * 16
                fidx[row, pl.ds(col, 16)] = flat
                vvals[row, pl.ds(col, 16)] = val
                return 0

            lax.fori_loop(0, CHS // 16, vv, 0)
            for k in range(6):
                fidx[31, pl.ds(32 + k * 16, 16)] = jnp.zeros((16,), jnp.int32)
                vvals[31, pl.ds(32 + k * 16, 16)] = jnp.zeros((16,),
                                                             jnp.float32)

            def piece(k, _):
                pltpu.sync_copy(vvals.at[k], sband.at[fidx.at[k]], add=True)
                return 0

            lax.fori_loop(0, 32, piece, 0)
            return 0

        lax.fori_loop(0, EPS // CHS, chunk, 0)
        plsc.subcore_barrier()

        def wout(i, _):
            pltpu.sync_copy(
                sband.at[pl.ds(pl.multiple_of(s * SHARE + i * ZW, 8), ZW)],
                stage)
            pltpu.sync_copy(
                stage,
                adj.at[pl.ds(pl.multiple_of(
                    band * BANDW + s * SHARE + i * ZW, 8), ZW)])
            return 0

        lax.fori_loop(0, SHARE // ZW, wout, 0)
        plsc.subcore_barrier()


def _s3_call(psrc, pdst):
    f = pl.kernel(
        _s3_body,
        out_type=jax.ShapeDtypeStruct((8 * BANDW,), jnp.float32),
        mesh=_mesh(),
        compiler_params=pltpu.CompilerParams(needs_layout_passes=False),
        scratch_types=[
            pltpu.VMEM((CHS,), jnp.int32),
            pltpu.VMEM((CHS,), jnp.int32),
            pltpu.VMEM((32, 128), jnp.int32),
            pltpu.VMEM((32, 128), jnp.float32),
            pltpu.VMEM((ZW,), jnp.float32),
            pltpu.VMEM((ZW,), jnp.float32),
            pltpu.VMEM_SHARED((BANDW,), jnp.float32),
        ],
        name="s3_adj_build",
    )
    return f(psrc, pdst)


RB = 1000


def _k1_body(x_ref, ind_ref, ux_ref, dinv_ref):
    dv = lax.rsqrt(ind_ref[...] + 1.0)
    ux_ref[...] = dv * x_ref[...]
    dinv_ref[...] = dv


def _k1_call(x, indeg2d):
    return pl.pallas_call(
        _k1_body,
        grid=(N // RB,),
        in_specs=[
            pl.BlockSpec((RB, FIN), lambda i: (i, 0)),
            pl.BlockSpec((RB, 1), lambda i: (i, 0)),
        ],
        out_specs=[
            pl.BlockSpec((RB, FIN), lambda i: (i, 0)),
            pl.BlockSpec((RB, 1), lambda i: (i, 0)),
        ],
        out_shape=[
            jax.ShapeDtypeStruct((N, FIN), jnp.float32),
            jax.ShapeDtypeStruct((N, 1), jnp.float32),
        ],
    )(x, indeg2d)


def _k2_body(aggx_ref, x_ref, dinv_ref, w_ref, b_ref, h_ref):
    dv = dinv_ref[...]
    t = dv * aggx_ref[...] + (dv * dv) * x_ref[...]
    h = jnp.dot(t, w_ref[...], preferred_element_type=jnp.float32)
    h_ref[...] = jnp.maximum(h + b_ref[...], 0.0)


def _k2_call(aggx, x, dinv, W1, b1):
    return pl.pallas_call(
        _k2_body,
        grid=(N // RB,),
        in_specs=[
            pl.BlockSpec((RB, FIN), lambda i: (i, 0)),
            pl.BlockSpec((RB, FIN), lambda i: (i, 0)),
            pl.BlockSpec((RB, 1), lambda i: (i, 0)),
            pl.BlockSpec((FIN, HID), lambda i: (0, 0)),
            pl.BlockSpec((1, HID), lambda i: (0, 0)),
        ],
        out_specs=pl.BlockSpec((RB, HID), lambda i: (i, 0)),
        out_shape=jax.ShapeDtypeStruct((N, HID), jnp.float32),
    )(aggx, x, dinv, W1, b1)


CCH = 256
AUXW = 8
NEG = -jnp.inf


_CONTRACT0 = (((0,), (0,)), ((), ()))


def _k3_body(h_ref, bcol_ref, ccol_ref, aux_ref,
             psum_ref, pmax_ref, bstat_ref, xps_ref, cstat_ref, bp_ref):
    i = pl.program_id(0)
    j = pl.program_id(1)
    h = h_ref[...]
    aux = aux_ref[...]
    ccol = ccol_ref[...]
    bcol = bcol_ref[...]
    iotaC = lax.broadcasted_iota(jnp.int32, (RB, CCH), 1) + i * CCH
    oneC = (iotaC == ccol).astype(jnp.float32)
    xps = lax.dot_general(oneC, h, _CONTRACT0,
                          preferred_element_type=jnp.float32)
    cst = lax.dot_general(oneC, aux, _CONTRACT0,
                          preferred_element_type=jnp.float32)
    btf = bcol.astype(jnp.float32)
    bpf = jnp.max(jnp.where(oneC > 0, btf, -1.0), axis=0, keepdims=True)

    @pl.when(j == 0)
    def _():
        xps_ref[...] = xps
        cstat_ref[...] = cst
        bp_ref[...] = bpf

    @pl.when(j > 0)
    def _():
        xps_ref[...] += xps
        cstat_ref[...] += cst
        bp_ref[...] = jnp.maximum(bp_ref[...], bpf)

    @pl.when(i == 0)
    def _():
        iotaB = lax.broadcasted_iota(jnp.int32, (RB, NB), 1)
        oneB = (iotaB == bcol).astype(jnp.float32)
        ps = lax.dot_general(oneB, h, _CONTRACT0,
                             preferred_element_type=jnp.float32)
        bs = lax.dot_general(oneB, aux, _CONTRACT0,
                             preferred_element_type=jnp.float32)

        @pl.when(j == 0)
        def _():
            psum_ref[...] = ps
            bstat_ref[...] = bs
            pmax_ref[...] = jnp.full((NB, HID), NEG, jnp.float32)

        @pl.when(j > 0)
        def _():
            psum_ref[...] += ps
            bstat_ref[...] += bs

        def sloop(sb, _):
            mk = bcol == sb
            mx = jnp.max(jnp.where(mk, h, NEG), axis=0, keepdims=True)
            pmax_ref[pl.ds(sb, 1), :] = jnp.maximum(
                pmax_ref[pl.ds(sb, 1), :], mx)
            return 0

        lax.fori_loop(0, NB, sloop, 0)


def _k3_call(h, bcol, ccol, aux):
    return pl.pallas_call(
        _k3_body,
        grid=(NC // CCH, N // RB),
        in_specs=[
            pl.BlockSpec((RB, HID), lambda i, j: (j, 0)),
            pl.BlockSpec((RB, 1), lambda i, j: (j, 0)),
            pl.BlockSpec((RB, 1), lambda i, j: (j, 0)),
            pl.BlockSpec((RB, AUXW), lambda i, j: (j, 0)),
        ],
        out_specs=[
            pl.BlockSpec((NB, HID), lambda i, j: (0, 0)),
            pl.BlockSpec((NB, HID), lambda i, j: (0, 0)),
            pl.BlockSpec((NB, AUXW), lambda i, j: (0, 0)),
            pl.BlockSpec((CCH, HID), lambda i, j: (i, 0)),
            pl.BlockSpec((CCH, AUXW), lambda i, j: (i, 0)),
            pl.BlockSpec((1, CCH), lambda i, j: (0, i)),
        ],
        out_shape=[
            jax.ShapeDtypeStruct((NB, HID), jnp.float32),
            jax.ShapeDtypeStruct((NB, HID), jnp.float32),
            jax.ShapeDtypeStruct((NB, AUXW), jnp.float32),
            jax.ShapeDtypeStruct((NC, HID), jnp.float32),
            jax.ShapeDtypeStruct((NC, AUXW), jnp.float32),
            jax.ShapeDtypeStruct((1, NC), jnp.float32),
        ],
    )(h, bcol, ccol, aux)


def _k5_body(xps_ref, cstat_ref, w_ref, b_ref, v_ref, u_ref, dv_ref):
    cnt = jnp.maximum(cstat_ref[:, 0:1], 1.0)
    xp = xps_ref[...] / cnt
    deg2 = cstat_ref[:, 1:2] + 1.0
    dv = lax.rsqrt(deg2)
    y = jnp.dot(xp, w_ref[...], preferred_element_type=jnp.float32)
    v_ref[...] = dv * y
    u_ref[...] = (dv * dv) * y + b_ref[...]
    dv_ref[...] = dv


def _k5_call(xp_sum, cstat, W2, b2):
    return pl.pallas_call(
        _k5_body,
        out_shape=[
            jax.ShapeDtypeStruct((NC, HID), jnp.float32),
            jax.ShapeDtypeStruct((NC, HID), jnp.float32),
            jax.ShapeDtypeStruct((NC, 1), jnp.float32),
        ],
    )(xp_sum, cstat, W2, b2)


def _k7_body(h2_ref, w_ref, b_ref, dv_ref, v_ref, u_ref):
    dv = dv_ref[...]
    y = jnp.dot(h2_ref[...], w_ref[...], preferred_element_type=jnp.float32)
    v_ref[...] = dv * y
    u_ref[...] = (dv * dv) * y + b_ref[...]


def _k7_call(h2, W3, b3, dinv2):
    return pl.pallas_call(
        _k7_body,
        out_shape=[
            jax.ShapeDtypeStruct((NC, HID), jnp.float32),
            jax.ShapeDtypeStruct((NC, HID), jnp.float32),
        ],
    )(h2, W3, b3, dinv2)


def _k6_body(adj_ref, v_ref, u_ref, dv_ref, out_ref):
    r = jnp.dot(adj_ref[...], v_ref[...], preferred_element_type=jnp.float32)
    out_ref[...] = jnp.maximum(dv_ref[...] * r + u_ref[...], 0.0)


def _k6_call(adj, v, u, dinv2):
    return pl.pallas_call(
        _k6_body,
        grid=(NC // BANDR,),
        in_specs=[
            pl.BlockSpec((BANDR, NC), lambda i: (i, 0)),
            pl.BlockSpec((NC, HID), lambda i: (0, 0)),
            pl.BlockSpec((BANDR, HID), lambda i: (i, 0)),
            pl.BlockSpec((BANDR, 1), lambda i: (i, 0)),
        ],
        out_specs=pl.BlockSpec((BANDR, HID), lambda i: (i, 0)),
        out_shape=jax.ShapeDtypeStruct((NC, HID), jnp.float32),
    )(adj, v, u, dinv2)


def _k8_body(h3_ref, bpr_ref, bpc_ref, psum_ref, pmax_ref, bstat_ref,
             wl1_ref, bl1_ref, wl2_ref, bl2_ref, out_ref, pmx_ref):
    h3 = h3_ref[...]
    bpr = jnp.clip(bpr_ref[...], 0.0, NB - 1.0)
    bpc = jnp.clip(bpc_ref[...], 0.0, NB - 1.0)
    iotaP = lax.broadcasted_iota(jnp.int32, (NB, NC), 0)
    oneP = (iotaP == bpr.astype(jnp.int32)).astype(jnp.float32)
    post_sum = jnp.dot(oneP, h3, preferred_element_type=jnp.float32)
    post_cnt = jnp.maximum(jnp.sum(oneP, axis=1, keepdims=True), 1.0)
    post_mean = post_sum / post_cnt

    def sloop(sb, _):
        mk = bpc == sb
        mx = jnp.max(jnp.where(mk, h3, NEG), axis=0, keepdims=True)
        pmx_ref[pl.ds(sb, 1), :] = mx
        return 0

    lax.fori_loop(0, NB, sloop, 0)
    post_max = pmx_ref[...]
    post_max = jnp.where(jnp.isfinite(post_max), post_max, 0.0)

    pre_cnt = jnp.maximum(bstat_ref[:, 0:1], 1.0)
    pre_mean = psum_ref[...] / pre_cnt
    pre_max = pmax_ref[...]
    pre_max = jnp.where(jnp.isfinite(pre_max), pre_max, 0.0)

    g = jnp.concatenate([pre_mean, pre_max, post_mean, post_max], axis=1)
    z1 = jnp.dot(g, wl1_ref[...], preferred_element_type=jnp.float32)
    z1 = jnp.maximum(z1 + bl1_ref[...], 0.0)
    z = jnp.dot(z1, wl2_ref[...], preferred_element_type=jnp.float32)
    z = z + bl2_ref[...]
    m = jnp.max(z, axis=1, keepdims=True)
    zs = z - m
    lse = jnp.log(jnp.sum(jnp.exp(zs), axis=1, keepdims=True))
    out_ref[...] = zs - lse


def _k8_call(h3, bp_row, bp_col, pre_sum, pre_max, bstat, Wl1, bl1, Wl2, bl2):
    return pl.pallas_call(
        _k8_body,
        out_shape=jax.ShapeDtypeStruct((NB, NCLS_PAD), jnp.float32),
        scratch_shapes=[pltpu.VMEM((NB, HID), jnp.float32)],
    )(h3, bp_row, bp_col, pre_sum, pre_max, bstat, Wl1, bl1, Wl2, bl2)


NCLS = 10
NCLS_PAD = 10


def kernel(x, edge_index, batch, cluster, num_clusters,
           W1, b1, W2, b2, W3, b3, Wl1, bl1, Wl2, bl2):
    src = edge_index[0].astype(jnp.int32)
    dst = edge_index[1].astype(jnp.int32)
    cluster = cluster.astype(jnp.int32)
    batch = batch.astype(jnp.int32)

    elist, counts, indeg_t, psrc, pdst = _s1_call(src, dst, cluster)
    indeg = indeg_t[:N]

    ux, dinv = _k1_call(x, indeg.reshape(N, 1))

    aggx_t = _s2_call(ux, elist, counts)
    aggx = aggx_t.reshape(NT * RPT, FIN)[:N]

    h = _k2_call(aggx, x, dinv, W1, b1.reshape(1, HID))

    aux = jnp.concatenate(
        [jnp.ones((N, 1), jnp.float32), indeg.reshape(N, 1),
         jnp.zeros((N, AUXW - 2), jnp.float32)], axis=1)
    bcol = batch.reshape(N, 1)
    ccol = cluster.reshape(N, 1)
    pre_sum, pre_max, bstat, xp_sum, cstat, bp = _k3_call(
        h, bcol, ccol, aux)

    adj = _s3_call(psrc, pdst).reshape(NC, NC)

    v2, u2, dinv2 = _k5_call(xp_sum, cstat, W2, b2.reshape(1, HID))
    h2 = _k6_call(adj, v2, u2, dinv2)
    v3, u3 = _k7_call(h2, W3, b3.reshape(1, HID), dinv2)
    h3 = _k6_call(adj, v3, u3, dinv2)

    out = _k8_call(h3, bp, bp.reshape(NC, 1), pre_sum, pre_max, bstat,
                   Wl1, bl1.reshape(1, 2 * HID), Wl2, bl2.reshape(1, NCLS))
    return out[:, :NCLS]


# speedup vs baseline: 2.7612x; 2.7612x over previous
import functools
import jax
import jax.numpy as jnp
from jax import lax
from jax.experimental import pallas as pl
from jax.experimental.pallas import tpu as pltpu
from jax.experimental.pallas import tpu_sc as plsc

N = 10000
E = 320000
FIN = 128
HID = 256
NB = 64
NC = 2560
NT = 32
RPT = 320
EPT = E // NT
CHA = 2000
CHB = 4000
CAP = E + 16384
G = 128
SENT = RPT << 14

@functools.cache
def _mesh():
    return plsc.VectorSubcoreMesh(core_axis_name="c", subcore_axis_name="s")


def _s1_body(src_hbm, dst_hbm, clus_hbm, elist, counts, indeg, psrc, pdst,
             sbuf0, sbuf1, dbuf0, dbuf1, packbuf, abuf_s, abuf_d, pbs, pbd,
             clus_v, hist, indeg_v, cnt_v, sem_b0, sem_b1):
    c = lax.axis_index("c")
    s = lax.axis_index("s")
    wid = s * 2 + c
    lane = lax.iota(jnp.int32, 16)
    ones16 = jnp.ones((16,), jnp.float32)
    sem_b = (sem_b0, sem_b1)
    sbufs = (sbuf0, sbuf1)
    dbufs = (dbuf0, dbuf1)

    pltpu.sync_copy(clus_hbm, clus_v)
    ebase = wid * EPT

    def chunk_a(a, _):
        off = pl.multiple_of(ebase + a * CHA, 8)
        pltpu.sync_copy(src_hbm.at[pl.ds(off, CHA)], abuf_s)
        pltpu.sync_copy(dst_hbm.at[pl.ds(off, CHA)], abuf_d)

        def va(v, _):
            s16 = abuf_s[pl.ds(v * 16, 16)]
            d16 = abuf_d[pl.ds(v * 16, 16)]
            pbs[pl.ds(v * 16, 16)] = plsc.load_gather(clus_v, [s16])
            pbd[pl.ds(v * 16, 16)] = plsc.load_gather(clus_v, [d16])
            return 0

        lax.fori_loop(0, CHA // 16, va, 0)
        pltpu.sync_copy(pbs, psrc.at[pl.ds(off, CHA)])
        pltpu.sync_copy(pbd, pdst.at[pl.ds(off, CHA)])
        return 0

    lax.fori_loop(0, EPT // CHA, chunk_a, 0)

    def zh(i, _):
        hist[pl.ds(i * 16, 16)] = jnp.zeros((16,), jnp.float32)
        return 0

    lax.fori_loop(0, (RPT * 16) // 16, zh, 0)

    lo = wid * RPT
    hi = lo + RPT

    def fetch_b(ch, p):
        off = pl.multiple_of(ch * CHB, 8)
        pltpu.async_copy(src_hbm.at[pl.ds(off, CHB)], sbufs[p], sem_b[p])
        pltpu.async_copy(dst_hbm.at[pl.ds(off, CHB)], dbufs[p], sem_b[p])

    def proc_b(ch, p, gbase):
        off = pl.multiple_of(ch * CHB, 8)
        pltpu.make_async_copy(src_hbm.at[pl.ds(off, CHB)], sbufs[p],
                              sem_b[p]).wait()
        pltpu.make_async_copy(dst_hbm.at[pl.ds(off, CHB)], dbufs[p],
                              sem_b[p]).wait()

        def vb(v, lcnt):
            s16 = sbufs[p][pl.ds(v * 16, 16)]
            d16 = dbufs[p][pl.ds(v * 16, 16)]
            m = (d16 >= lo) & (d16 < hi)
            ld16 = d16 - lo
            packed = s16 | (ld16 << 14)
            plsc.store_compressed(packbuf.at[pl.ds(lcnt, 16)], packed, mask=m)
            hidx = jnp.where(m, lane * RPT + ld16, lane * RPT)
            plsc.addupdate_scatter(hist, [hidx], ones16, mask=m)
            return lcnt + plsc.all_reduce_population_count(m)[0]

        lcnt = lax.fori_loop(0, CHB // 16, vb, jnp.int32(0))
        pad = (8 - (lcnt & 7)) & 7
        pm = lane < pad
        plsc.store_compressed(packbuf.at[pl.ds(lcnt, 16)],
                              jnp.full((16,), SENT, jnp.int32), mask=pm)
        pltpu.sync_copy(packbuf,
                        elist.at[pl.ds(pl.multiple_of(wid * CAP + gbase, 8),
                                       CHB + 16)])
        return gbase + lcnt + pad

    NPB = E // CHB // 2

    fetch_b(0, 0)

    def pair_b(q, gbase):
        fetch_b(2 * q + 1, 1)
        gbase = proc_b(2 * q, 0, gbase)

        @pl.when(q < NPB - 1)
        def _():
            fetch_b(2 * q + 2, 0)

        return proc_b(2 * q + 1, 1, gbase)

    gfinal = lax.fori_loop(0, NPB, pair_b, jnp.int32(0))

    def fs(i, _):
        packbuf[pl.ds(i * 16, 16)] = jnp.full((16,), SENT, jnp.int32)
        return 0

    lax.fori_loop(0, (CHB + 16) // 16, fs, 0)
    pltpu.sync_copy(packbuf,
                    elist.at[pl.ds(pl.multiple_of(wid * CAP + gfinal, 8),
                                   CHB + 16)])
    pltpu.sync_copy(packbuf,
                    elist.at[pl.ds(pl.multiple_of(
                        wid * CAP + gfinal + CHB + 16, 8), CHB + 16)])

    cnt_v[...] = jnp.full((16,), gfinal, jnp.int32)
    pltpu.sync_copy(cnt_v.at[pl.ds(0, 8)],
                    counts.at[pl.ds(pl.multiple_of(wid * 8, 8), 8)])

    def hr(v, _):
        acc = hist[pl.ds(v * 16, 16)]
        for l in range(1, 16):
            acc = acc + hist[pl.ds(l * RPT + v * 16, 16)]
        indeg_v[pl.ds(v * 16, 16)] = acc
        return 0

    lax.fori_loop(0, RPT // 16, hr, 0)
    pltpu.sync_copy(indeg_v,
                    indeg.at[pl.ds(pl.multiple_of(wid * RPT, 8), RPT)])


def _s1_call(src, dst, cluster):
    f = pl.kernel(
        _s1_body,
        out_type=(
            jax.ShapeDtypeStruct((NT * CAP,), jnp.int32),
            jax.ShapeDtypeStruct((NT * 8,), jnp.int32),
            jax.ShapeDtypeStruct((NT * RPT,), jnp.float32),
            jax.ShapeDtypeStruct((E,), jnp.int32),
            jax.ShapeDtypeStruct((E,), jnp.int32),
        ),
        mesh=_mesh(),
        compiler_params=pltpu.CompilerParams(needs_layout_passes=False),
        scratch_types=[
            pltpu.VMEM((CHB,), jnp.int32),
            pltpu.VMEM((CHB,), jnp.int32),
            pltpu.VMEM((CHB,), jnp.int32),
            pltpu.VMEM((CHB,), jnp.int32),
            pltpu.VMEM((CHB + 16,), jnp.int32),
            pltpu.VMEM((CHA,), jnp.int32),
            pltpu.VMEM((CHA,), jnp.int32),
            pltpu.VMEM((CHA,), jnp.int32),
            pltpu.VMEM((CHA,), jnp.int32),
            pltpu.VMEM((N,), jnp.int32),
            pltpu.VMEM((RPT * 16,), jnp.float32),
            pltpu.VMEM((RPT,), jnp.float32),
            pltpu.VMEM((16,), jnp.int32),
            pltpu.SemaphoreType.DMA,
            pltpu.SemaphoreType.DMA,
        ],
        name="s1_edge_scan",
    )
    return f(src, dst, cluster)


ACCW = (RPT + 8) * FIN


def _s2_body(ux_hbm, elist, counts, aggx, acc, ebuf0, idx0, ldb0,
             rows00, cnt_v, sem00):
    c = lax.axis_index("c")
    s = lax.axis_index("s")
    wid = s * 2 + c

    def za(i, _):
        acc[pl.ds(i * 16, 16)] = jnp.zeros((16,), jnp.float32)
        return 0

    lax.fori_loop(0, ACCW // 16, za, 0)

    pltpu.sync_copy(counts.at[pl.ds(pl.multiple_of(wid * 8, 8), 8)],
                    cnt_v.at[pl.ds(0, 8)])
    cnt = cnt_v[pl.ds(0, 16)][0]

    nch = (cnt + (G - 1)) // G

    def chunk(g, _):
        base = wid * CAP + g * G
        pltpu.sync_copy(
            elist.at[pl.ds(pl.multiple_of(base, 8), G)], ebuf0)
        for v in range(G // 16):
            e16 = ebuf0[pl.ds(v * 16, 16)]
            idx0[pl.ds(v * 16, 16)] = e16 & 0x3FFF
            ldb0[pl.ds(v * 16, 16)] = (e16 >> 7) & 0x7FFF80
        pltpu.async_copy(ux_hbm.at[idx0], rows00, sem00).wait()

        def edge(v, _):
            ld16 = ldb0[pl.ds(v * 16, 16)]
            for k in range(16):
                base = ld16[k]
                i = v * 16 + k
                rs = [rows00[i, pl.ds(j * 16, 16)] for j in range(FIN // 16)]
                accs = [acc[pl.ds(base + j * 16, 16)]
                        for j in range(FIN // 16)]
                for j in range(FIN // 16):
                    acc[pl.ds(base + j * 16, 16)] = accs[j] + rs[j]
            return 0

        lax.fori_loop(0, G // 16, edge, 0)
        return 0

    lax.fori_loop(0, nch, chunk, 0)

    pltpu.sync_copy(acc.at[pl.ds(0, RPT * FIN)],
                    aggx.at[pl.ds(pl.multiple_of(wid * RPT * FIN, 8),
                                  RPT * FIN)])


def _s2_call(ux, elist, counts):
    f = pl.kernel(
        _s2_body,
        out_type=jax.ShapeDtypeStruct((NT * RPT * FIN,), jnp.float32),
        mesh=_mesh(),
        compiler_params=pltpu.CompilerParams(needs_layout_passes=False),
        scratch_types=[
            pltpu.VMEM((ACCW,), jnp.float32),
            pltpu.VMEM((G,), jnp.int32),
            pltpu.VMEM((G,), jnp.int32),
            pltpu.VMEM((G,), jnp.int32),
            pltpu.VMEM((G, FIN), jnp.float32),
            pltpu.VMEM((16,), jnp.int32),
            pltpu.SemaphoreType.DMA,
        ],
        name="s2_gather_acc",
    )
    return f(ux, elist, counts)


BANDR = 320
BANDW = BANDR * NC
EPS = E // 16
CHS = 4000
ZW = 6400
SHARE = BANDW // 16


def _s3_body(psrc_hbm, pdst_hbm, adj, pbuf, dbuf, fidx, vvals, zbuf, stage,
             sband):
    c = lax.axis_index("c")
    s = lax.axis_index("s")

    def zz(i, _):
        zbuf[pl.ds(i * 16, 16)] = jnp.zeros((16,), jnp.float32)
        return 0

    lax.fori_loop(0, ZW // 16, zz, 0)

    for b in range(4):
        band = c * 4 + b
        lo = band * BANDR

        def zb(i, _):
            pltpu.sync_copy(
                zbuf,
                sband.at[pl.ds(pl.multiple_of(s * SHARE + i * ZW, 8), ZW)])
            return 0

        lax.fori_loop(0, SHARE // ZW, zb, 0)
        plsc.subcore_barrier()

        ebase = s * EPS

        def chunk(ch, _):
            off = pl.multiple_of(ebase + ch * CHS, 8)
            pltpu.sync_copy(psrc_hbm.at[pl.ds(off, CHS)], pbuf)
            pltpu.sync_copy(pdst_hbm.at[pl.ds(off, CHS)], dbuf)

            def vv(vi, _):
                ps = pbuf[pl.ds(vi * 16, 16)]
                pd = dbuf[pl.ds(vi * 16, 16)]
                m = (pd >= lo) & (pd < lo + BANDR)
                flat = jnp.where(m, (pd - lo) * NC + ps, ps)
                val = jnp.where(m, 1.0, 0.0).astype(jnp.float32)
                row = vi // 8
                col = (vi % 8) * 16
                fidx[row, pl.ds(col, 16)] = flat
                vvals[row, pl.ds(col, 16)] = val
                return 0

            lax.fori_loop(0, CHS // 16, vv, 0)
            for k in range(6):
                fidx[31, pl.ds(32 + k * 16, 16)] = jnp.zeros((16,), jnp.int32)
                vvals[31, pl.ds(32 + k * 16, 16)] = jnp.zeros((16,),
                                                             jnp.float32)

            def piece(k, _):
                pltpu.sync_copy(vvals.at[k], sband.at[fidx.at[k]], add=True)
                return 0

            lax.fori_loop(0, 32, piece, 0)
            return 0

        lax.fori_loop(0, EPS // CHS, chunk, 0)
        plsc.subcore_barrier()

        def wout(i, _):
            pltpu.sync_copy(
                sband.at[pl.ds(pl.multiple_of(s * SHARE + i * ZW, 8), ZW)],
                stage)
            pltpu.sync_copy(
                stage,
                adj.at[pl.ds(pl.multiple_of(
                    band * BANDW + s * SHARE + i * ZW, 8), ZW)])
            return 0

        lax.fori_loop(0, SHARE // ZW, wout, 0)
        plsc.subcore_barrier()


def _s3_call(psrc, pdst):
    f = pl.kernel(
        _s3_body,
        out_type=jax.ShapeDtypeStruct((8 * BANDW,), jnp.float32),
        mesh=_mesh(),
        compiler_params=pltpu.CompilerParams(needs_layout_passes=False),
        scratch_types=[
            pltpu.VMEM((CHS,), jnp.int32),
            pltpu.VMEM((CHS,), jnp.int32),
            pltpu.VMEM((32, 128), jnp.int32),
            pltpu.VMEM((32, 128), jnp.float32),
            pltpu.VMEM((ZW,), jnp.float32),
            pltpu.VMEM((ZW,), jnp.float32),
            pltpu.VMEM_SHARED((BANDW,), jnp.float32),
        ],
        name="s3_adj_build",
    )
    return f(psrc, pdst)


RB = 1000


def _k1_body(x_ref, ind_ref, ux_ref, dinv_ref):
    dv = lax.rsqrt(ind_ref[...] + 1.0)
    ux_ref[...] = dv * x_ref[...]
    dinv_ref[...] = dv


def _k1_call(x, indeg2d):
    return pl.pallas_call(
        _k1_body,
        grid=(N // RB,),
        in_specs=[
            pl.BlockSpec((RB, FIN), lambda i: (i, 0)),
            pl.BlockSpec((RB, 1), lambda i: (i, 0)),
        ],
        out_specs=[
            pl.BlockSpec((RB, FIN), lambda i: (i, 0)),
            pl.BlockSpec((RB, 1), lambda i: (i, 0)),
        ],
        out_shape=[
            jax.ShapeDtypeStruct((N, FIN), jnp.float32),
            jax.ShapeDtypeStruct((N, 1), jnp.float32),
        ],
    )(x, indeg2d)


def _k2_body(aggx_ref, x_ref, dinv_ref, w_ref, b_ref, h_ref):
    dv = dinv_ref[...]
    t = dv * aggx_ref[...] + (dv * dv) * x_ref[...]
    h = jnp.dot(t, w_ref[...], preferred_element_type=jnp.float32)
    h_ref[...] = jnp.maximum(h + b_ref[...], 0.0)


def _k2_call(aggx, x, dinv, W1, b1):
    return pl.pallas_call(
        _k2_body,
        grid=(N // RB,),
        in_specs=[
            pl.BlockSpec((RB, FIN), lambda i: (i, 0)),
            pl.BlockSpec((RB, FIN), lambda i: (i, 0)),
            pl.BlockSpec((RB, 1), lambda i: (i, 0)),
            pl.BlockSpec((FIN, HID), lambda i: (0, 0)),
            pl.BlockSpec((1, HID), lambda i: (0, 0)),
        ],
        out_specs=pl.BlockSpec((RB, HID), lambda i: (i, 0)),
        out_shape=jax.ShapeDtypeStruct((N, HID), jnp.float32),
    )(aggx, x, dinv, W1, b1)


CCH = 256
AUXW = 8
NEG = -jnp.inf


_CONTRACT0 = (((0,), (0,)), ((), ()))


def _k3_body(h_ref, bcol_ref, ccol_ref, aux_ref,
             psum_ref, pmax_ref, bstat_ref, xps_ref, cstat_ref, bp_ref):
    i = pl.program_id(0)
    j = pl.program_id(1)
    h = h_ref[...]
    aux = aux_ref[...]
    ccol = ccol_ref[...]
    bcol = bcol_ref[...]
    iotaC = lax.broadcasted_iota(jnp.int32, (RB, CCH), 1) + i * CCH
    oneC = (iotaC == ccol).astype(jnp.float32)
    xps = lax.dot_general(oneC, h, _CONTRACT0,
                          preferred_element_type=jnp.float32)
    cst = lax.dot_general(oneC, aux, _CONTRACT0,
                          preferred_element_type=jnp.float32)
    btf = bcol.astype(jnp.float32)
    bpf = jnp.max(jnp.where(oneC > 0, btf, -1.0), axis=0, keepdims=True)

    @pl.when(j == 0)
    def _():
        xps_ref[...] = xps
        cstat_ref[...] = cst
        bp_ref[...] = bpf

    @pl.when(j > 0)
    def _():
        xps_ref[...] += xps
        cstat_ref[...] += cst
        bp_ref[...] = jnp.maximum(bp_ref[...], bpf)

    @pl.when(i == 0)
    def _():
        iotaB = lax.broadcasted_iota(jnp.int32, (RB, NB), 1)
        oneB = (iotaB == bcol).astype(jnp.float32)
        ps = lax.dot_general(oneB, h, _CONTRACT0,
                             preferred_element_type=jnp.float32)
        bs = lax.dot_general(oneB, aux, _CONTRACT0,
                             preferred_element_type=jnp.float32)

        @pl.when(j == 0)
        def _():
            psum_ref[...] = ps
            bstat_ref[...] = bs
            pmax_ref[...] = jnp.full((NB, HID), NEG, jnp.float32)

        @pl.when(j > 0)
        def _():
            psum_ref[...] += ps
            bstat_ref[...] += bs

        def sloop(sb, _):
            mk = bcol == sb
            mx = jnp.max(jnp.where(mk, h, NEG), axis=0, keepdims=True)
            pmax_ref[pl.ds(sb, 1), :] = jnp.maximum(
                pmax_ref[pl.ds(sb, 1), :], mx)
            return 0

        lax.fori_loop(0, NB, sloop, 0)


def _k3_call(h, bcol, ccol, aux):
    return pl.pallas_call(
        _k3_body,
        grid=(NC // CCH, N // RB),
        in_specs=[
            pl.BlockSpec((RB, HID), lambda i, j: (j, 0)),
            pl.BlockSpec((RB, 1), lambda i, j: (j, 0)),
            pl.BlockSpec((RB, 1), lambda i, j: (j, 0)),
            pl.BlockSpec((RB, AUXW), lambda i, j: (j, 0)),
        ],
        out_specs=[
            pl.BlockSpec((NB, HID), lambda i, j: (0, 0)),
            pl.BlockSpec((NB, HID), lambda i, j: (0, 0)),
            pl.BlockSpec((NB, AUXW), lambda i, j: (0, 0)),
            pl.BlockSpec((CCH, HID), lambda i, j: (i, 0)),
            pl.BlockSpec((CCH, AUXW), lambda i, j: (i, 0)),
            pl.BlockSpec((1, CCH), lambda i, j: (0, i)),
        ],
        out_shape=[
            jax.ShapeDtypeStruct((NB, HID), jnp.float32),
            jax.ShapeDtypeStruct((NB, HID), jnp.float32),
            jax.ShapeDtypeStruct((NB, AUXW), jnp.float32),
            jax.ShapeDtypeStruct((NC, HID), jnp.float32),
            jax.ShapeDtypeStruct((NC, AUXW), jnp.float32),
            jax.ShapeDtypeStruct((1, NC), jnp.float32),
        ],
    )(h, bcol, ccol, aux)


def _k5_body(xps_ref, cstat_ref, w_ref, b_ref, v_ref, u_ref, dv_ref):
    cnt = jnp.maximum(cstat_ref[:, 0:1], 1.0)
    xp = xps_ref[...] / cnt
    deg2 = cstat_ref[:, 1:2] + 1.0
    dv = lax.rsqrt(deg2)
    y = jnp.dot(xp, w_ref[...], preferred_element_type=jnp.float32)
    v_ref[...] = dv * y
    u_ref[...] = (dv * dv) * y + b_ref[...]
    dv_ref[...] = dv


def _k5_call(xp_sum, cstat, W2, b2):
    return pl.pallas_call(
        _k5_body,
        out_shape=[
            jax.ShapeDtypeStruct((NC, HID), jnp.float32),
            jax.ShapeDtypeStruct((NC, HID), jnp.float32),
            jax.ShapeDtypeStruct((NC, 1), jnp.float32),
        ],
    )(xp_sum, cstat, W2, b2)


def _k7_body(h2_ref, w_ref, b_ref, dv_ref, v_ref, u_ref):
    dv = dv_ref[...]
    y = jnp.dot(h2_ref[...], w_ref[...], preferred_element_type=jnp.float32)
    v_ref[...] = dv * y
    u_ref[...] = (dv * dv) * y + b_ref[...]


def _k7_call(h2, W3, b3, dinv2):
    return pl.pallas_call(
        _k7_body,
        out_shape=[
            jax.ShapeDtypeStruct((NC, HID), jnp.float32),
            jax.ShapeDtypeStruct((NC, HID), jnp.float32),
        ],
    )(h2, W3, b3, dinv2)


def _k6_body(adj_ref, v_ref, u_ref, dv_ref, out_ref):
    r = jnp.dot(adj_ref[...], v_ref[...], preferred_element_type=jnp.float32)
    out_ref[...] = jnp.maximum(dv_ref[...] * r + u_ref[...], 0.0)


def _k6_call(adj, v, u, dinv2):
    return pl.pallas_call(
        _k6_body,
        grid=(NC // BANDR,),
        in_specs=[
            pl.BlockSpec((BANDR, NC), lambda i: (i, 0)),
            pl.BlockSpec((NC, HID), lambda i: (0, 0)),
            pl.BlockSpec((BANDR, HID), lambda i: (i, 0)),
            pl.BlockSpec((BANDR, 1), lambda i: (i, 0)),
        ],
        out_specs=pl.BlockSpec((BANDR, HID), lambda i: (i, 0)),
        out_shape=jax.ShapeDtypeStruct((NC, HID), jnp.float32),
    )(adj, v, u, dinv2)


def _k8_body(h3_ref, bpr_ref, bpc_ref, psum_ref, pmax_ref, bstat_ref,
             wl1_ref, bl1_ref, wl2_ref, bl2_ref, out_ref, pmx_ref):
    h3 = h3_ref[...]
    bpr = jnp.clip(bpr_ref[...], 0.0, NB - 1.0)
    bpc = jnp.clip(bpc_ref[...], 0.0, NB - 1.0)
    iotaP = lax.broadcasted_iota(jnp.int32, (NB, NC), 0)
    oneP = (iotaP == bpr.astype(jnp.int32)).astype(jnp.float32)
    post_sum = jnp.dot(oneP, h3, preferred_element_type=jnp.float32)
    post_cnt = jnp.maximum(jnp.sum(oneP, axis=1, keepdims=True), 1.0)
    post_mean = post_sum / post_cnt

    def sloop(sb, _):
        mk = bpc == sb
        mx = jnp.max(jnp.where(mk, h3, NEG), axis=0, keepdims=True)
        pmx_ref[pl.ds(sb, 1), :] = mx
        return 0

    lax.fori_loop(0, NB, sloop, 0)
    post_max = pmx_ref[...]
    post_max = jnp.where(jnp.isfinite(post_max), post_max, 0.0)

    pre_cnt = jnp.maximum(bstat_ref[:, 0:1], 1.0)
    pre_mean = psum_ref[...] / pre_cnt
    pre_max = pmax_ref[...]
    pre_max = jnp.where(jnp.isfinite(pre_max), pre_max, 0.0)

    g = jnp.concatenate([pre_mean, pre_max, post_mean, post_max], axis=1)
    z1 = jnp.dot(g, wl1_ref[...], preferred_element_type=jnp.float32)
    z1 = jnp.maximum(z1 + bl1_ref[...], 0.0)
    z = jnp.dot(z1, wl2_ref[...], preferred_element_type=jnp.float32)
    z = z + bl2_ref[...]
    m = jnp.max(z, axis=1, keepdims=True)
    zs = z - m
    lse = jnp.log(jnp.sum(jnp.exp(zs), axis=1, keepdims=True))
    out_ref[...] = zs - lse


def _k8_call(h3, bp_row, bp_col, pre_sum, pre_max, bstat, Wl1, bl1, Wl2, bl2):
    return pl.pallas_call(
        _k8_body,
        out_shape=jax.ShapeDtypeStruct((NB, NCLS_PAD), jnp.float32),
        scratch_shapes=[pltpu.VMEM((NB, HID), jnp.float32)],
    )(h3, bp_row, bp_col, pre_sum, pre_max, bstat, Wl1, bl1, Wl2, bl2)


NCLS = 10
NCLS_PAD = 10


def kernel(x, edge_index, batch, cluster, num_clusters,
           W1, b1, W2, b2, W3, b3, Wl1, bl1, Wl2, bl2):
    src = edge_index[0].astype(jnp.int32)
    dst = edge_index[1].astype(jnp.int32)
    cluster = cluster.astype(jnp.int32)
    batch = batch.astype(jnp.int32)

    elist, counts, indeg_t, psrc, pdst = _s1_call(src, dst, cluster)
    indeg = indeg_t[:N]

    ux, dinv = _k1_call(x, indeg.reshape(N, 1))

    aggx_t = _s2_call(ux, elist, counts)
    aggx = aggx_t.reshape(NT * RPT, FIN)[:N]

    h = _k2_call(aggx, x, dinv, W1, b1.reshape(1, HID))

    aux = jnp.concatenate(
        [jnp.ones((N, 1), jnp.float32), indeg.reshape(N, 1),
         jnp.zeros((N, AUXW - 2), jnp.float32)], axis=1)
    bcol = batch.reshape(N, 1)
    ccol = cluster.reshape(N, 1)
    pre_sum, pre_max, bstat, xp_sum, cstat, bp = _k3_call(
        h, bcol, ccol, aux)

    adj = _s3_call(psrc, pdst).reshape(NC, NC)

    v2, u2, dinv2 = _k5_call(xp_sum, cstat, W2, b2.reshape(1, HID))
    h2 = _k6_call(adj, v2, u2, dinv2)
    v3, u3 = _k7_call(h2, W3, b3.reshape(1, HID), dinv2)
    h3 = _k6_call(adj, v3, u3, dinv2)

    out = _k8_call(h3, bp, bp.reshape(NC, 1), pre_sum, pre_max, bstat,
                   Wl1, bl1.reshape(1, 2 * HID), Wl2, bl2.reshape(1, NCLS))
    return out[:, :NCLS]
